# baseline (speedup 1.0000x reference)
import functools

import jax
import jax.numpy as jnp
from jax import lax
from jax.experimental import pallas as pl
from jax.experimental.pallas import tpu as pltpu
from jax.experimental.pallas import tpu_sc as plsc

D_MODEL = 1024
N_HEADS = 8
D_KEYS = 128
HALF = D_KEYS // 2
N_EXPERTS = 256
K = 8
NTOK = 256


def _topk_t(s, k, payload=None):
    c, t = s.shape
    row = lax.broadcasted_iota(jnp.int32, (c, t), 0)
    vals, idxs, pays = [], [], []
    cur = s
    for _ in range(k):
        m = jnp.max(cur, axis=0, keepdims=True)
        hit = cur == m
        idx = jnp.min(jnp.where(hit, row, c), axis=0, keepdims=True)
        sel = row == idx
        vals.append(m)
        idxs.append(idx)
        if payload is not None:
            pays.append(jnp.sum(jnp.where(sel, payload, 0), axis=0, keepdims=True))
        cur = jnp.where(sel, -jnp.inf, cur)
    out_v = jnp.concatenate(vals, axis=0)
    out_i = jnp.concatenate(idxs, axis=0)
    if payload is not None:
        return out_v, out_i, jnp.concatenate(pays, axis=0)
    return out_v, out_i


def _routing_body(x_ref, wq_ref, bq_ref, keys_ref, gate_ref, idx_ref):
    x = x_ref[...]
    q = lax.dot_general(x, wq_ref[...], (((1,), (1,)), ((), ())),
                        preferred_element_type=jnp.float32)
    q = q + bq_ref[...][None, :]
    for h in range(N_HEADS):
        q1 = q[:, h * D_KEYS: h * D_KEYS + HALF]
        q2 = q[:, h * D_KEYS + HALF: (h + 1) * D_KEYS]
        k1 = keys_ref[h, 0]
        k2 = keys_ref[h, 1]
        s1 = lax.dot_general(k1, q1, (((1,), (1,)), ((), ())),
                             preferred_element_type=jnp.float32)
        s2 = lax.dot_general(k2, q2, (((1,), (1,)), ((), ())),
                             preferred_element_type=jnp.float32)
        sv1, iv1 = _topk_t(s1, K)
        sv2, iv2 = _topk_t(s2, K)
        all_s = jnp.concatenate([sv1[i:i + 1] + sv2 for i in range(K)], axis=0)
        all_i = jnp.concatenate([iv1[i:i + 1] * N_EXPERTS + iv2 for i in range(K)],
                                axis=0)
        sc, _, sel_i = _topk_t(all_s, K, payload=all_i)
        m = jnp.max(sc, axis=0, keepdims=True)
        e = jnp.exp(sc - m)
        g = e / jnp.sum(e, axis=0, keepdims=True)
        gate_ref[h] = g
        idx_ref[h] = sel_i


def _routing(x, Wq, bq, keys):
    return pl.pallas_call(
        _routing_body,
        out_shape=[
            jax.ShapeDtypeStruct((N_HEADS, K, NTOK), jnp.float32),
            jax.ShapeDtypeStruct((N_HEADS, K, NTOK), jnp.int32),
        ],
    )(x, Wq, bq, keys)


_NW = 32
_ROWS = N_HEADS * NTOK * K
_NSPLIT = 4
_HROWS = _ROWS // _NSPLIT
_PER_W = _HROWS // _NW
_CHUNK = 32
_NCH = _PER_W // _CHUNK


def _gather_kernel_body(idx_hbm, down_hbm, up_hbm, out_down, out_up,
                        idx_v, buf0, buf1, buf2, gsem, wsem):
    wid = lax.axis_index("s") * 2 + lax.axis_index("c")
    base = wid * _PER_W
    pltpu.sync_copy(idx_hbm.at[pl.ds(base, _PER_W)], idx_v)
    bufs = (buf0, buf1, buf2)
    steps = [(table, out, i)
             for table, out in ((down_hbm, out_down), (up_hbm, out_up))
             for i in range(_NCH)]
    ghs, whs = [], []

    def _fire(step):
        table, out, i = steps[step]
        if step >= 3:
            whs[step - 3].wait()
        ghs.append(pltpu.async_copy(
            table.at[idx_v.at[pl.ds(i * _CHUNK, _CHUNK)]],
            bufs[step % 3], gsem))

    def _drain(step):
        table, out, i = steps[step]
        ghs[step].wait()
        whs.append(pltpu.async_copy(
            bufs[step % 3], out.at[pl.ds(base + i * _CHUNK, _CHUNK)], wsem))

    n_steps = len(steps)
    for step in range(n_steps):
        _fire(step)
        if step >= 2:
            _drain(step - 2)
    _drain(n_steps - 2)
    _drain(n_steps - 1)
    for wh in whs[-3:]:
        wh.wait()


@functools.cache
def _make_gather():
    return functools.partial(
        pl.kernel,
        mesh=plsc.VectorSubcoreMesh(core_axis_name="c", subcore_axis_name="s"),
        out_type=[
            jax.ShapeDtypeStruct((_HROWS, D_MODEL), jnp.float32),
            jax.ShapeDtypeStruct((_HROWS, D_MODEL), jnp.float32),
        ],
        scratch_types=[
            pltpu.VMEM((_PER_W,), jnp.int32),
            pltpu.VMEM((_CHUNK, D_MODEL), jnp.float32),
            pltpu.VMEM((_CHUNK, D_MODEL), jnp.float32),
            pltpu.VMEM((_CHUNK, D_MODEL), jnp.float32),
            pltpu.SemaphoreType.DMA,
            pltpu.SemaphoreType.DMA,
        ],
    )(_gather_kernel_body)


def _gather(idx_flat, down, up):
    return _make_gather()(idx_flat, down, up)


_JBLK = 512
_NJH = _HROWS // _JBLK


def _ffn_step(x_ref, wd_ref, wu_ref, g_ref, acc_ref):
    xb = x_ref[...].astype(jnp.bfloat16)
    wd = wd_ref[...].astype(jnp.bfloat16)
    xc = lax.dot_general(xb, wd, (((1,), (1,)), ((), ())),
                         preferred_element_type=jnp.float32)
    xc = jnp.maximum(xc, 0.0) * g_ref[0]
    wu = wu_ref[...].astype(jnp.bfloat16)
    acc_ref[...] += lax.dot_general(xc.astype(jnp.bfloat16), wu,
                                    (((1,), (0,)), ((), ())),
                                    preferred_element_type=jnp.float32)


def _ffn_first_body(x_ref, wd_ref, wu_ref, g_ref, o_ref, acc_ref):
    j = pl.program_id(0)

    @pl.when(j == 0)
    def _():
        acc_ref[...] = jnp.zeros_like(acc_ref)

    _ffn_step(x_ref, wd_ref, wu_ref, g_ref, acc_ref)

    @pl.when(j == _NJH - 1)
    def _():
        o_ref[...] = acc_ref[...]


def _ffn_chain_body(x_ref, wd_ref, wu_ref, g_ref, p_ref, o_ref, acc_ref):
    j = pl.program_id(0)

    @pl.when(j == 0)
    def _():
        acc_ref[...] = p_ref[...]

    _ffn_step(x_ref, wd_ref, wu_ref, g_ref, acc_ref)

    @pl.when(j == _NJH - 1)
    def _():
        o_ref[...] = acc_ref[...]


_FFN_SPECS = [
    pl.BlockSpec((NTOK, D_MODEL), lambda j: (0, 0)),
    pl.BlockSpec((_JBLK, D_MODEL), lambda j: (j, 0)),
    pl.BlockSpec((_JBLK, D_MODEL), lambda j: (j, 0)),
    pl.BlockSpec((1, 1, _JBLK), lambda j: (j, 0, 0)),
]


def _ffn_first(x, wd, wu, gate3):
    return pl.pallas_call(
        _ffn_first_body,
        grid=(_NJH,),
        in_specs=_FFN_SPECS,
        out_specs=pl.BlockSpec((NTOK, D_MODEL), lambda j: (0, 0)),
        out_shape=jax.ShapeDtypeStruct((NTOK, D_MODEL), jnp.float32),
        scratch_shapes=[pltpu.VMEM((NTOK, D_MODEL), jnp.float32)],
    )(x, wd, wu, gate3)


def _ffn_chain(x, wd, wu, gate3, part):
    return pl.pallas_call(
        _ffn_chain_body,
        grid=(_NJH,),
        in_specs=_FFN_SPECS + [pl.BlockSpec((NTOK, D_MODEL), lambda j: (0, 0))],
        out_specs=pl.BlockSpec((NTOK, D_MODEL), lambda j: (0, 0)),
        out_shape=jax.ShapeDtypeStruct((NTOK, D_MODEL), jnp.float32),
        scratch_shapes=[pltpu.VMEM((NTOK, D_MODEL), jnp.float32)],
    )(x, wd, wu, gate3, part)


def kernel(queries, Wq, bq, keys, w_down_embed, w_up_embed):
    n, t, d = queries.shape
    x = queries.reshape(-1, d)
    gate, idx = _routing(x, Wq, bq, keys)
    idx_flat = idx.reshape(-1)
    gate3 = gate.reshape(_ROWS // _JBLK, 1, _JBLK)
    nh = _ROWS // _JBLK // _NSPLIT
    out = None
    for s in range(_NSPLIT):
        wd, wu = _gather(idx_flat[s * _HROWS:(s + 1) * _HROWS],
                         w_down_embed, w_up_embed)
        g = gate3[s * nh:(s + 1) * nh]
        if out is None:
            out = _ffn_first(x, wd, wu, g)
        else:
            out = _ffn_chain(x, wd, wu, g, out)
    return out.reshape(n, t, d)


# speedup vs baseline: 1.0162x; 1.0162x over previous
import functools

import jax
import jax.numpy as jnp
from jax import lax
from jax.experimental import pallas as pl
from jax.experimental.pallas import tpu as pltpu
from jax.experimental.pallas import tpu_sc as plsc

D_MODEL = 1024
N_HEADS = 8
D_KEYS = 128
HALF = D_KEYS // 2
N_EXPERTS = 256
K = 8
NTOK = 256


def _topk_t(s, k, payload=None):
    c, t = s.shape
    row = lax.broadcasted_iota(jnp.int32, (c, t), 0)
    vals, idxs, pays = [], [], []
    cur = s
    for _ in range(k):
        m = jnp.max(cur, axis=0, keepdims=True)
        hit = cur == m
        idx = jnp.min(jnp.where(hit, row, c), axis=0, keepdims=True)
        sel = row == idx
        vals.append(m)
        idxs.append(idx)
        if payload is not None:
            pays.append(jnp.sum(jnp.where(sel, payload, 0), axis=0, keepdims=True))
        cur = jnp.where(sel, -jnp.inf, cur)
    out_v = jnp.concatenate(vals, axis=0)
    out_i = jnp.concatenate(idxs, axis=0)
    if payload is not None:
        return out_v, out_i, jnp.concatenate(pays, axis=0)
    return out_v, out_i


def _routing_body(x_ref, wq_ref, bq_ref, keys_ref, gate_ref, idx_ref):
    x = x_ref[...]
    q = lax.dot_general(x, wq_ref[...], (((1,), (1,)), ((), ())),
                        preferred_element_type=jnp.float32)
    q = q + bq_ref[...][None, :]
    for h in range(N_HEADS):
        q1 = q[:, h * D_KEYS: h * D_KEYS + HALF]
        q2 = q[:, h * D_KEYS + HALF: (h + 1) * D_KEYS]
        k1 = keys_ref[h, 0]
        k2 = keys_ref[h, 1]
        s1 = lax.dot_general(k1, q1, (((1,), (1,)), ((), ())),
                             preferred_element_type=jnp.float32)
        s2 = lax.dot_general(k2, q2, (((1,), (1,)), ((), ())),
                             preferred_element_type=jnp.float32)
        sv1, iv1 = _topk_t(s1, K)
        sv2, iv2 = _topk_t(s2, K)
        all_s = jnp.concatenate([sv1[i:i + 1] + sv2 for i in range(K)], axis=0)
        all_i = jnp.concatenate([iv1[i:i + 1] * N_EXPERTS + iv2 for i in range(K)],
                                axis=0)
        sc, _, sel_i = _topk_t(all_s, K, payload=all_i)
        m = jnp.max(sc, axis=0, keepdims=True)
        e = jnp.exp(sc - m)
        g = e / jnp.sum(e, axis=0, keepdims=True)
        gate_ref[h] = g
        idx_ref[h] = sel_i


def _routing(x, Wq, bq, keys):
    return pl.pallas_call(
        _routing_body,
        out_shape=[
            jax.ShapeDtypeStruct((N_HEADS, K, NTOK), jnp.float32),
            jax.ShapeDtypeStruct((N_HEADS, K, NTOK), jnp.int32),
        ],
    )(x, Wq, bq, keys)


_NW = 32
_ROWS = N_HEADS * NTOK * K
_SPLITS = (8192, 6144, 2048)
_CHUNK = 32


def _gather_body(hrows, idx_hbm, down_hbm, up_hbm, out_down, out_up,
                 idx_v, buf0, buf1, buf2, gsem, wsem):
    per_w = hrows // _NW
    nch = per_w // _CHUNK
    wid = lax.axis_index("s") * 2 + lax.axis_index("c")
    base = wid * per_w
    pltpu.sync_copy(idx_hbm.at[pl.ds(base, per_w)], idx_v)
    bufs = (buf0, buf1, buf2)
    steps = [(table, out, i)
             for table, out in ((down_hbm, out_down), (up_hbm, out_up))
             for i in range(nch)]
    ghs, whs = [], []

    def _fire(step):
        table, out, i = steps[step]
        if step >= 3:
            whs[step - 3].wait()
        ghs.append(pltpu.async_copy(
            table.at[idx_v.at[pl.ds(i * _CHUNK, _CHUNK)]],
            bufs[step % 3], gsem))

    def _drain(step):
        table, out, i = steps[step]
        ghs[step].wait()
        whs.append(pltpu.async_copy(
            bufs[step % 3], out.at[pl.ds(base + i * _CHUNK, _CHUNK)], wsem))

    n_steps = len(steps)
    for step in range(n_steps):
        _fire(step)
        if step >= 2:
            _drain(step - 2)
    _drain(n_steps - 2)
    _drain(n_steps - 1)
    for wh in whs[-3:]:
        wh.wait()


@functools.cache
def _make_gather(hrows):
    return functools.partial(
        pl.kernel,
        mesh=plsc.VectorSubcoreMesh(core_axis_name="c", subcore_axis_name="s"),
        out_type=[
            jax.ShapeDtypeStruct((hrows, D_MODEL), jnp.float32),
            jax.ShapeDtypeStruct((hrows, D_MODEL), jnp.float32),
        ],
        scratch_types=[
            pltpu.VMEM((hrows // _NW,), jnp.int32),
            pltpu.VMEM((_CHUNK, D_MODEL), jnp.float32),
            pltpu.VMEM((_CHUNK, D_MODEL), jnp.float32),
            pltpu.VMEM((_CHUNK, D_MODEL), jnp.float32),
            pltpu.SemaphoreType.DMA,
            pltpu.SemaphoreType.DMA,
        ],
    )(functools.partial(_gather_body, hrows))


def _gather(idx_flat, down, up):
    return _make_gather(idx_flat.shape[0])(idx_flat, down, up)


_JBLK = 512


def _ffn_step(x_ref, wd_ref, wu_ref, g_ref, acc_ref):
    xb = x_ref[...].astype(jnp.bfloat16)
    wd = wd_ref[...].astype(jnp.bfloat16)
    xc = lax.dot_general(xb, wd, (((1,), (1,)), ((), ())),
                         preferred_element_type=jnp.float32)
    xc = jnp.maximum(xc, 0.0) * g_ref[0]
    wu = wu_ref[...].astype(jnp.bfloat16)
    acc_ref[...] += lax.dot_general(xc.astype(jnp.bfloat16), wu,
                                    (((1,), (0,)), ((), ())),
                                    preferred_element_type=jnp.float32)


def _ffn_first_body(x_ref, wd_ref, wu_ref, g_ref, o_ref, acc_ref):
    j = pl.program_id(0)

    @pl.when(j == 0)
    def _():
        acc_ref[...] = jnp.zeros_like(acc_ref)

    _ffn_step(x_ref, wd_ref, wu_ref, g_ref, acc_ref)

    @pl.when(j == pl.num_programs(0) - 1)
    def _():
        o_ref[...] = acc_ref[...]


def _ffn_chain_body(x_ref, wd_ref, wu_ref, g_ref, p_ref, o_ref, acc_ref):
    j = pl.program_id(0)

    @pl.when(j == 0)
    def _():
        acc_ref[...] = p_ref[...]

    _ffn_step(x_ref, wd_ref, wu_ref, g_ref, acc_ref)

    @pl.when(j == pl.num_programs(0) - 1)
    def _():
        o_ref[...] = acc_ref[...]


_FFN_SPECS = [
    pl.BlockSpec((NTOK, D_MODEL), lambda j: (0, 0)),
    pl.BlockSpec((_JBLK, D_MODEL), lambda j: (j, 0)),
    pl.BlockSpec((_JBLK, D_MODEL), lambda j: (j, 0)),
    pl.BlockSpec((1, 1, _JBLK), lambda j: (j, 0, 0)),
]


def _ffn_first(x, wd, wu, gate3):
    return pl.pallas_call(
        _ffn_first_body,
        grid=(wd.shape[0] // _JBLK,),
        in_specs=_FFN_SPECS,
        out_specs=pl.BlockSpec((NTOK, D_MODEL), lambda j: (0, 0)),
        out_shape=jax.ShapeDtypeStruct((NTOK, D_MODEL), jnp.float32),
        scratch_shapes=[pltpu.VMEM((NTOK, D_MODEL), jnp.float32)],
    )(x, wd, wu, gate3)


def _ffn_chain(x, wd, wu, gate3, part):
    return pl.pallas_call(
        _ffn_chain_body,
        grid=(wd.shape[0] // _JBLK,),
        in_specs=_FFN_SPECS + [pl.BlockSpec((NTOK, D_MODEL), lambda j: (0, 0))],
        out_specs=pl.BlockSpec((NTOK, D_MODEL), lambda j: (0, 0)),
        out_shape=jax.ShapeDtypeStruct((NTOK, D_MODEL), jnp.float32),
        scratch_shapes=[pltpu.VMEM((NTOK, D_MODEL), jnp.float32)],
    )(x, wd, wu, gate3, part)


def kernel(queries, Wq, bq, keys, w_down_embed, w_up_embed):
    n, t, d = queries.shape
    x = queries.reshape(-1, d)
    gate, idx = _routing(x, Wq, bq, keys)
    idx_flat = idx.reshape(-1)
    gate3 = gate.reshape(_ROWS // _JBLK, 1, _JBLK)
    out = None
    off = 0
    for hrows in _SPLITS:
        wd, wu = _gather(idx_flat[off:off + hrows], w_down_embed, w_up_embed)
        g = gate3[off // _JBLK:(off + hrows) // _JBLK]
        if out is None:
            out = _ffn_first(x, wd, wu, g)
        else:
            out = _ffn_chain(x, wd, wu, g, out)
        off += hrows
    return out.reshape(n, t, d)


# speedup vs baseline: 1.0186x; 1.0024x over previous
import functools

import jax
import jax.numpy as jnp
from jax import lax
from jax.experimental import pallas as pl
from jax.experimental.pallas import tpu as pltpu
from jax.experimental.pallas import tpu_sc as plsc

D_MODEL = 1024
N_HEADS = 8
D_KEYS = 128
HALF = D_KEYS // 2
N_EXPERTS = 256
K = 8
NTOK = 256


def _topk_t(s, k, payload=None):
    c, t = s.shape
    row = lax.broadcasted_iota(jnp.int32, (c, t), 0)
    vals, idxs, pays = [], [], []
    cur = s
    for _ in range(k):
        m = jnp.max(cur, axis=0, keepdims=True)
        hit = cur == m
        idx = jnp.min(jnp.where(hit, row, c), axis=0, keepdims=True)
        sel = row == idx
        vals.append(m)
        idxs.append(idx)
        if payload is not None:
            pays.append(jnp.sum(jnp.where(sel, payload, 0), axis=0, keepdims=True))
        cur = jnp.where(sel, -jnp.inf, cur)
    out_v = jnp.concatenate(vals, axis=0)
    out_i = jnp.concatenate(idxs, axis=0)
    if payload is not None:
        return out_v, out_i, jnp.concatenate(pays, axis=0)
    return out_v, out_i


def _routing_body(x_ref, wq_ref, bq_ref, keys_ref, gate_ref, idx_ref):
    x = x_ref[...]
    q = lax.dot_general(x, wq_ref[...], (((1,), (1,)), ((), ())),
                        preferred_element_type=jnp.float32)
    q = q + bq_ref[...][None, :]
    for h in range(N_HEADS):
        q1 = q[:, h * D_KEYS: h * D_KEYS + HALF]
        q2 = q[:, h * D_KEYS + HALF: (h + 1) * D_KEYS]
        k1 = keys_ref[h, 0]
        k2 = keys_ref[h, 1]
        s1 = lax.dot_general(k1, q1, (((1,), (1,)), ((), ())),
                             preferred_element_type=jnp.float32)
        s2 = lax.dot_general(k2, q2, (((1,), (1,)), ((), ())),
                             preferred_element_type=jnp.float32)
        sv1, iv1 = _topk_t(s1, K)
        sv2, iv2 = _topk_t(s2, K)
        all_s = jnp.concatenate([sv1[i:i + 1] + sv2 for i in range(K)], axis=0)
        all_i = jnp.concatenate([iv1[i:i + 1] * N_EXPERTS + iv2 for i in range(K)],
                                axis=0)
        sc, _, sel_i = _topk_t(all_s, K, payload=all_i)
        m = jnp.max(sc, axis=0, keepdims=True)
        e = jnp.exp(sc - m)
        g = e / jnp.sum(e, axis=0, keepdims=True)
        gate_ref[h] = g
        idx_ref[h] = sel_i


def _routing(x, Wq, bq, keys):
    return pl.pallas_call(
        _routing_body,
        out_shape=[
            jax.ShapeDtypeStruct((N_HEADS, K, NTOK), jnp.float32),
            jax.ShapeDtypeStruct((N_HEADS, K, NTOK), jnp.int32),
        ],
    )(x, Wq, bq, keys)


_NW = 32
_ROWS = N_HEADS * NTOK * K
_SPLITS = (12288, 4096)
_CHUNK = 32


def _gather_body(hrows, idx_hbm, down_hbm, up_hbm, out_down, out_up,
                 idx_v, buf0, buf1, buf2, gsem, wsem):
    per_w = hrows // _NW
    nch = per_w // _CHUNK
    wid = lax.axis_index("s") * 2 + lax.axis_index("c")
    base = wid * per_w
    pltpu.sync_copy(idx_hbm.at[pl.ds(base, per_w)], idx_v)
    bufs = (buf0, buf1, buf2)
    steps = [(table, out, i)
             for table, out in ((down_hbm, out_down), (up_hbm, out_up))
             for i in range(nch)]
    ghs, whs = [], []

    def _fire(step):
        table, out, i = steps[step]
        if step >= 3:
            whs[step - 3].wait()
        ghs.append(pltpu.async_copy(
            table.at[idx_v.at[pl.ds(i * _CHUNK, _CHUNK)]],
            bufs[step % 3], gsem))

    def _drain(step):
        table, out, i = steps[step]
        ghs[step].wait()
        whs.append(pltpu.async_copy(
            bufs[step % 3], out.at[pl.ds(base + i * _CHUNK, _CHUNK)], wsem))

    n_steps = len(steps)
    for step in range(n_steps):
        _fire(step)
        if step >= 2:
            _drain(step - 2)
    _drain(n_steps - 2)
    _drain(n_steps - 1)
    for wh in whs[-3:]:
        wh.wait()


@functools.cache
def _make_gather(hrows):
    return functools.partial(
        pl.kernel,
        mesh=plsc.VectorSubcoreMesh(core_axis_name="c", subcore_axis_name="s"),
        out_type=[
            jax.ShapeDtypeStruct((hrows, D_MODEL), jnp.float32),
            jax.ShapeDtypeStruct((hrows, D_MODEL), jnp.float32),
        ],
        scratch_types=[
            pltpu.VMEM((hrows // _NW,), jnp.int32),
            pltpu.VMEM((_CHUNK, D_MODEL), jnp.float32),
            pltpu.VMEM((_CHUNK, D_MODEL), jnp.float32),
            pltpu.VMEM((_CHUNK, D_MODEL), jnp.float32),
            pltpu.SemaphoreType.DMA,
            pltpu.SemaphoreType.DMA,
        ],
    )(functools.partial(_gather_body, hrows))


def _gather(idx_flat, down, up):
    return _make_gather(idx_flat.shape[0])(idx_flat, down, up)


_JBLK = 512


def _ffn_step(x_ref, wd_ref, wu_ref, g_ref, acc_ref):
    xb = x_ref[...].astype(jnp.bfloat16)
    wd = wd_ref[...].astype(jnp.bfloat16)
    xc = lax.dot_general(xb, wd, (((1,), (1,)), ((), ())),
                         preferred_element_type=jnp.float32)
    xc = jnp.maximum(xc, 0.0) * g_ref[0]
    wu = wu_ref[...].astype(jnp.bfloat16)
    acc_ref[...] += lax.dot_general(xc.astype(jnp.bfloat16), wu,
                                    (((1,), (0,)), ((), ())),
                                    preferred_element_type=jnp.float32)


def _ffn_first_body(x_ref, wd_ref, wu_ref, g_ref, o_ref, acc_ref):
    j = pl.program_id(0)

    @pl.when(j == 0)
    def _():
        acc_ref[...] = jnp.zeros_like(acc_ref)

    _ffn_step(x_ref, wd_ref, wu_ref, g_ref, acc_ref)

    @pl.when(j == pl.num_programs(0) - 1)
    def _():
        o_ref[...] = acc_ref[...]


def _ffn_chain_body(x_ref, wd_ref, wu_ref, g_ref, p_ref, o_ref, acc_ref):
    j = pl.program_id(0)

    @pl.when(j == 0)
    def _():
        acc_ref[...] = p_ref[...]

    _ffn_step(x_ref, wd_ref, wu_ref, g_ref, acc_ref)

    @pl.when(j == pl.num_programs(0) - 1)
    def _():
        o_ref[...] = acc_ref[...]


_FFN_SPECS = [
    pl.BlockSpec((NTOK, D_MODEL), lambda j: (0, 0)),
    pl.BlockSpec((_JBLK, D_MODEL), lambda j: (j, 0)),
    pl.BlockSpec((_JBLK, D_MODEL), lambda j: (j, 0)),
    pl.BlockSpec((1, 1, _JBLK), lambda j: (j, 0, 0)),
]


def _ffn_first(x, wd, wu, gate3):
    return pl.pallas_call(
        _ffn_first_body,
        grid=(wd.shape[0] // _JBLK,),
        in_specs=_FFN_SPECS,
        out_specs=pl.BlockSpec((NTOK, D_MODEL), lambda j: (0, 0)),
        out_shape=jax.ShapeDtypeStruct((NTOK, D_MODEL), jnp.float32),
        scratch_shapes=[pltpu.VMEM((NTOK, D_MODEL), jnp.float32)],
    )(x, wd, wu, gate3)


def _ffn_chain(x, wd, wu, gate3, part):
    return pl.pallas_call(
        _ffn_chain_body,
        grid=(wd.shape[0] // _JBLK,),
        in_specs=_FFN_SPECS + [pl.BlockSpec((NTOK, D_MODEL), lambda j: (0, 0))],
        out_specs=pl.BlockSpec((NTOK, D_MODEL), lambda j: (0, 0)),
        out_shape=jax.ShapeDtypeStruct((NTOK, D_MODEL), jnp.float32),
        scratch_shapes=[pltpu.VMEM((NTOK, D_MODEL), jnp.float32)],
    )(x, wd, wu, gate3, part)


def kernel(queries, Wq, bq, keys, w_down_embed, w_up_embed):
    n, t, d = queries.shape
    x = queries.reshape(-1, d)
    gate, idx = _routing(x, Wq, bq, keys)
    idx_flat = idx.reshape(-1)
    gate3 = gate.reshape(_ROWS // _JBLK, 1, _JBLK)
    out = None
    off = 0
    for hrows in _SPLITS:
        wd, wu = _gather(idx_flat[off:off + hrows], w_down_embed, w_up_embed)
        g = gate3[off // _JBLK:(off + hrows) // _JBLK]
        if out is None:
            out = _ffn_first(x, wd, wu, g)
        else:
            out = _ffn_chain(x, wd, wu, g, out)
        off += hrows
    return out.reshape(n, t, d)


# speedup vs baseline: 1.0506x; 1.0313x over previous
import functools

import jax
import jax.numpy as jnp
from jax import lax
from jax.experimental import pallas as pl
from jax.experimental.pallas import tpu as pltpu
from jax.experimental.pallas import tpu_sc as plsc

D_MODEL = 1024
N_HEADS = 8
D_KEYS = 128
HALF = D_KEYS // 2
N_EXPERTS = 256
K = 8
NTOK = 256


def _topk_t(s, k, payload=None):
    c, t = s.shape
    row = lax.broadcasted_iota(jnp.int32, (c, t), 0)
    vals, idxs, pays = [], [], []
    cur = s
    for _ in range(k):
        m = jnp.max(cur, axis=0, keepdims=True)
        hit = cur == m
        idx = jnp.min(jnp.where(hit, row, c), axis=0, keepdims=True)
        sel = row == idx
        vals.append(m)
        idxs.append(idx)
        if payload is not None:
            pays.append(jnp.sum(jnp.where(sel, payload, 0), axis=0, keepdims=True))
        cur = jnp.where(sel, -jnp.inf, cur)
    out_v = jnp.concatenate(vals, axis=0)
    out_i = jnp.concatenate(idxs, axis=0)
    if payload is not None:
        return out_v, out_i, jnp.concatenate(pays, axis=0)
    return out_v, out_i


def _routing_body(x_ref, wq_ref, bq_ref, keys_ref, gate_ref, idx_ref):
    x = x_ref[...]
    q = lax.dot_general(x, wq_ref[...], (((1,), (1,)), ((), ())),
                        preferred_element_type=jnp.float32)
    q = q + bq_ref[...][None, :]
    for h in range(N_HEADS):
        q1 = q[:, h * D_KEYS: h * D_KEYS + HALF]
        q2 = q[:, h * D_KEYS + HALF: (h + 1) * D_KEYS]
        k1 = keys_ref[h, 0]
        k2 = keys_ref[h, 1]
        s1 = lax.dot_general(k1, q1, (((1,), (1,)), ((), ())),
                             preferred_element_type=jnp.float32)
        s2 = lax.dot_general(k2, q2, (((1,), (1,)), ((), ())),
                             preferred_element_type=jnp.float32)
        sv1, iv1 = _topk_t(s1, K)
        sv2, iv2 = _topk_t(s2, K)
        all_s = jnp.concatenate([sv1[i:i + 1] + sv2 for i in range(K)], axis=0)
        all_i = jnp.concatenate([iv1[i:i + 1] * N_EXPERTS + iv2 for i in range(K)],
                                axis=0)
        sc, _, sel_i = _topk_t(all_s, K, payload=all_i)
        m = jnp.max(sc, axis=0, keepdims=True)
        e = jnp.exp(sc - m)
        g = e / jnp.sum(e, axis=0, keepdims=True)
        gate_ref[h] = g
        idx_ref[h] = sel_i


def _routing(x, Wq, bq, keys):
    return pl.pallas_call(
        _routing_body,
        out_shape=[
            jax.ShapeDtypeStruct((N_HEADS, K, NTOK), jnp.float32),
            jax.ShapeDtypeStruct((N_HEADS, K, NTOK), jnp.int32),
        ],
    )(x, Wq, bq, keys)


_NW = 32
_ROWS = N_HEADS * NTOK * K
_SPLITS = (8192, 8192)
_CHUNK = 32


def _gather_body(hrows, idx_hbm, down_hbm, up_hbm, out_down, out_up,
                 idx_v, buf0, buf1, buf2, gsem, wsem):
    per_w = hrows // _NW
    nch = per_w // _CHUNK
    wid = lax.axis_index("s") * 2 + lax.axis_index("c")
    base = wid * per_w
    pltpu.sync_copy(idx_hbm.at[pl.ds(base, per_w)], idx_v)
    bufs = (buf0, buf1, buf2)
    steps = [(table, out, i)
             for table, out in ((down_hbm, out_down), (up_hbm, out_up))
             for i in range(nch)]
    ghs, whs = [], []

    def _fire(step):
        table, out, i = steps[step]
        if step >= 3:
            whs[step - 3].wait()
        ghs.append(pltpu.async_copy(
            table.at[idx_v.at[pl.ds(i * _CHUNK, _CHUNK)]],
            bufs[step % 3], gsem))

    def _drain(step):
        table, out, i = steps[step]
        ghs[step].wait()
        whs.append(pltpu.async_copy(
            bufs[step % 3], out.at[pl.ds(base + i * _CHUNK, _CHUNK)], wsem))

    n_steps = len(steps)
    for step in range(n_steps):
        _fire(step)
        if step >= 2:
            _drain(step - 2)
    _drain(n_steps - 2)
    _drain(n_steps - 1)
    for wh in whs[-3:]:
        wh.wait()


@functools.cache
def _make_gather(hrows):
    return functools.partial(
        pl.kernel,
        mesh=plsc.VectorSubcoreMesh(core_axis_name="c", subcore_axis_name="s"),
        out_type=[
            jax.ShapeDtypeStruct((hrows, D_MODEL), jnp.float32),
            jax.ShapeDtypeStruct((hrows, D_MODEL), jnp.float32),
        ],
        scratch_types=[
            pltpu.VMEM((hrows // _NW,), jnp.int32),
            pltpu.VMEM((_CHUNK, D_MODEL), jnp.float32),
            pltpu.VMEM((_CHUNK, D_MODEL), jnp.float32),
            pltpu.VMEM((_CHUNK, D_MODEL), jnp.float32),
            pltpu.SemaphoreType.DMA,
            pltpu.SemaphoreType.DMA,
        ],
    )(functools.partial(_gather_body, hrows))


def _gather(idx_flat, down, up):
    return _make_gather(idx_flat.shape[0])(idx_flat, down, up)


_JBLK = 1024


def _ffn_step(x_ref, wd_ref, wu_ref, g_ref, acc_ref):
    xb = x_ref[...].astype(jnp.bfloat16)
    wd = wd_ref[...].astype(jnp.bfloat16)
    xc = lax.dot_general(xb, wd, (((1,), (1,)), ((), ())),
                         preferred_element_type=jnp.float32)
    xc = jnp.maximum(xc, 0.0) * g_ref[0]
    wu = wu_ref[...].astype(jnp.bfloat16)
    acc_ref[...] += lax.dot_general(xc.astype(jnp.bfloat16), wu,
                                    (((1,), (0,)), ((), ())),
                                    preferred_element_type=jnp.float32)


def _ffn_first_body(x_ref, wd_ref, wu_ref, g_ref, o_ref, acc_ref):
    j = pl.program_id(0)

    @pl.when(j == 0)
    def _():
        acc_ref[...] = jnp.zeros_like(acc_ref)

    _ffn_step(x_ref, wd_ref, wu_ref, g_ref, acc_ref)

    @pl.when(j == pl.num_programs(0) - 1)
    def _():
        o_ref[...] = acc_ref[...]


def _ffn_chain_body(x_ref, wd_ref, wu_ref, g_ref, p_ref, o_ref, acc_ref):
    j = pl.program_id(0)

    @pl.when(j == 0)
    def _():
        acc_ref[...] = p_ref[...]

    _ffn_step(x_ref, wd_ref, wu_ref, g_ref, acc_ref)

    @pl.when(j == pl.num_programs(0) - 1)
    def _():
        o_ref[...] = acc_ref[...]


_FFN_SPECS = [
    pl.BlockSpec((NTOK, D_MODEL), lambda j: (0, 0)),
    pl.BlockSpec((_JBLK, D_MODEL), lambda j: (j, 0)),
    pl.BlockSpec((_JBLK, D_MODEL), lambda j: (j, 0)),
    pl.BlockSpec((1, 1, _JBLK), lambda j: (j, 0, 0)),
]


def _ffn_first(x, wd, wu, gate3):
    return pl.pallas_call(
        _ffn_first_body,
        grid=(wd.shape[0] // _JBLK,),
        in_specs=_FFN_SPECS,
        out_specs=pl.BlockSpec((NTOK, D_MODEL), lambda j: (0, 0)),
        out_shape=jax.ShapeDtypeStruct((NTOK, D_MODEL), jnp.float32),
        scratch_shapes=[pltpu.VMEM((NTOK, D_MODEL), jnp.float32)],
    )(x, wd, wu, gate3)


def _ffn_chain(x, wd, wu, gate3, part):
    return pl.pallas_call(
        _ffn_chain_body,
        grid=(wd.shape[0] // _JBLK,),
        in_specs=_FFN_SPECS + [pl.BlockSpec((NTOK, D_MODEL), lambda j: (0, 0))],
        out_specs=pl.BlockSpec((NTOK, D_MODEL), lambda j: (0, 0)),
        out_shape=jax.ShapeDtypeStruct((NTOK, D_MODEL), jnp.float32),
        scratch_shapes=[pltpu.VMEM((NTOK, D_MODEL), jnp.float32)],
    )(x, wd, wu, gate3, part)


def kernel(queries, Wq, bq, keys, w_down_embed, w_up_embed):
    n, t, d = queries.shape
    x = queries.reshape(-1, d)
    gate, idx = _routing(x, Wq, bq, keys)
    idx_flat = idx.reshape(-1)
    gate3 = gate.reshape(_ROWS // _JBLK, 1, _JBLK)
    out = None
    off = 0
    for hrows in _SPLITS:
        wd, wu = _gather(idx_flat[off:off + hrows], w_down_embed, w_up_embed)
        g = gate3[off // _JBLK:(off + hrows) // _JBLK]
        if out is None:
            out = _ffn_first(x, wd, wu, g)
        else:
            out = _ffn_chain(x, wd, wu, g, out)
        off += hrows
    return out.reshape(n, t, d)
